# baseline (speedup 1.0000x reference)
import functools

import jax
import jax.numpy as jnp
from jax import lax
from jax.experimental import pallas as pl
from jax.experimental.pallas import tpu as pltpu
from jax.experimental.pallas import tpu_sc as plsc

NC = 2
NS = 16
NW = NC * NS
REP = 8
NBUF = 4
CK = 16


def _emb_kernel_body(n, tok_len, ctx_len, d,
                     idx_hbm, table_hbm, ctx8_hbm, out_hbm,
                     idx_v, gbufs, ctx8_v, gsems, wsems, csem):
    rpw = n // NW
    nck = rpw // CK
    nq = tok_len * nck
    ncw = ctx_len * (rpw // REP)
    nm = nq // NBUF
    cw_per = -(-ncw // nm)

    wid = lax.axis_index("s") * NC + lax.axis_index("c")
    c0 = wid * rpw

    for g in range(tok_len):
        pltpu.sync_copy(idx_hbm.at[pl.ds(g * n + c0, rpw)],
                        idx_v.at[pl.ds(g * rpw, rpw)])
    pltpu.sync_copy(ctx8_hbm, ctx8_v)

    def g_src(q):
        g = q // nck
        i = q % nck
        return table_hbm.at[idx_v.at[pl.ds(g * rpw + i * CK, CK)]]

    def w_dst(q):
        g = q // nck
        i = q % nck
        t = jnp.where(g == 0, 0, g + ctx_len)
        return out_hbm.at[t, pl.ds(c0 + i * CK, CK), :]

    def ctx_src(j):
        return ctx8_v.at[pl.ds((j // (rpw // REP)) * REP, REP)]

    def ctx_dst(j):
        k = j // (rpw // REP)
        i = j % (rpw // REP)
        return out_hbm.at[k + 1, pl.ds(c0 + i * REP, REP), :]

    for b in range(NBUF):
        pltpu.async_copy(g_src(b), gbufs[b], gsems[b])

    def outer(m, carry):
        q0 = m * NBUF
        for b in range(NBUF):
            q = q0 + b
            pltpu.make_async_copy(g_src(q), gbufs[b], gsems[b]).wait()
            pltpu.async_copy(gbufs[b], w_dst(q), wsems[b])
        for s in range(cw_per):
            j = m * cw_per + s

            @pl.when(j < ncw)
            def _():
                pltpu.async_copy(ctx_src(j), ctx_dst(j), csem)
        for b in range(NBUF):
            q = q0 + b
            qn = q + NBUF

            @pl.when(qn < nq)
            def _():
                pltpu.make_async_copy(gbufs[b], w_dst(q), wsems[b]).wait()
                pltpu.async_copy(g_src(qn), gbufs[b], gsems[b])
        return carry

    lax.fori_loop(0, nm, outer, 0)

    for b in range(NBUF):
        q = nq - NBUF + b
        pltpu.make_async_copy(gbufs[b], w_dst(q), wsems[b]).wait()

    def ctx_drain(j, carry):
        pltpu.make_async_copy(ctx_src(j), ctx_dst(j), csem).wait()
        return carry

    lax.fori_loop(0, ncw, ctx_drain, 0)


def _mask_body(m_ref, o_ref, *, ctx_len):
    n = m_ref.shape[0]
    ones = jnp.ones((n, ctx_len), dtype=m_ref.dtype)
    o_ref[...] = jnp.concatenate([ones, m_ref[...]], axis=1)


def kernel(class_token_ids, class_attention_mask, embedding_table, context_vectors_text):
    n, tok_len = class_token_ids.shape
    ctx_len, d = context_vectors_text.shape
    seq = 1 + ctx_len + (tok_len - 1)
    rpw = n // NW

    ids_t = class_token_ids.T.reshape(-1)
    ctx8 = jnp.repeat(context_vectors_text, REP, axis=0)

    mesh = plsc.VectorSubcoreMesh(core_axis_name="c", subcore_axis_name="s")
    emb_t = pl.kernel(
        functools.partial(_emb_kernel_body, n, tok_len, ctx_len, d),
        out_type=jax.ShapeDtypeStruct((seq, n, d), jnp.float32),
        mesh=mesh,
        scratch_types=[
            pltpu.VMEM((tok_len * rpw,), jnp.int32),
            [pltpu.VMEM((CK, d), jnp.float32) for _ in range(NBUF)],
            pltpu.VMEM((REP * ctx_len, d), jnp.float32),
            [pltpu.SemaphoreType.DMA for _ in range(NBUF)],
            [pltpu.SemaphoreType.DMA for _ in range(NBUF)],
            pltpu.SemaphoreType.DMA,
        ],
        compiler_params=pltpu.CompilerParams(use_tc_tiling_on_sc=True),
    )(ids_t, embedding_table, ctx8)

    mask = pl.pallas_call(
        functools.partial(_mask_body, ctx_len=ctx_len),
        out_shape=jax.ShapeDtypeStruct((n, ctx_len + tok_len), class_attention_mask.dtype),
    )(class_attention_mask)

    return emb_t.transpose(1, 0, 2), mask


# speedup vs baseline: 1.0236x; 1.0236x over previous
import functools

import jax
import jax.numpy as jnp
from jax import lax
from jax.experimental import pallas as pl
from jax.experimental.pallas import tpu as pltpu
from jax.experimental.pallas import tpu_sc as plsc

NC = 2
NS = 16
NW = NC * NS
REP = 8
NBUF = 2
CK = 32


def _emb_kernel_body(n, tok_len, ctx_len, d,
                     idx_hbm, table_hbm, ctx8_hbm, out_hbm,
                     idx_v, gbufs, ctx8_v, gsems, wsems, csem):
    rpw = n // NW
    nck = rpw // CK
    nq = tok_len * nck
    ncw = ctx_len * (rpw // REP)
    nm = nq // NBUF
    cw_per = -(-ncw // nm)

    wid = lax.axis_index("s") * NC + lax.axis_index("c")
    c0 = wid * rpw

    pltpu.sync_copy(idx_hbm.at[pl.ds(wid * tok_len * rpw, tok_len * rpw)], idx_v)
    pltpu.sync_copy(ctx8_hbm, ctx8_v)

    def g_src(q):
        g = q // nck
        i = q % nck
        return table_hbm.at[idx_v.at[pl.ds(g * rpw + i * CK, CK)]]

    def w_dst(q):
        g = q // nck
        i = q % nck
        t = jnp.where(g == 0, 0, g + ctx_len)
        return out_hbm.at[t, pl.ds(c0 + i * CK, CK), :]

    def ctx_src(j):
        return ctx8_v.at[pl.ds((j // (rpw // REP)) * REP, REP)]

    def ctx_dst(j):
        k = j // (rpw // REP)
        i = j % (rpw // REP)
        return out_hbm.at[k + 1, pl.ds(c0 + i * REP, REP), :]

    for b in range(NBUF):
        pltpu.async_copy(g_src(b), gbufs[b], gsems[b])

    def outer(m, carry):
        q0 = m * NBUF
        for b in range(NBUF):
            q = q0 + b
            pltpu.make_async_copy(g_src(q), gbufs[b], gsems[b]).wait()
            pltpu.async_copy(gbufs[b], w_dst(q), wsems[b])
        for s in range(cw_per):
            j = m * cw_per + s

            @pl.when(j < ncw)
            def _():
                pltpu.async_copy(ctx_src(j), ctx_dst(j), csem)
        for b in range(NBUF):
            q = q0 + b
            qn = q + NBUF

            @pl.when(qn < nq)
            def _():
                pltpu.make_async_copy(gbufs[b], w_dst(q), wsems[b]).wait()
                pltpu.async_copy(g_src(qn), gbufs[b], gsems[b])
        return carry

    lax.fori_loop(0, nm, outer, 0)

    for b in range(NBUF):
        q = nq - NBUF + b
        pltpu.make_async_copy(gbufs[b], w_dst(q), wsems[b]).wait()

    def ctx_drain(j, carry):
        pltpu.make_async_copy(ctx_src(j), ctx_dst(j), csem).wait()
        return carry

    lax.fori_loop(0, ncw, ctx_drain, 0)


def _mask_body(m_ref, o_ref, *, ctx_len):
    n = m_ref.shape[0]
    ones = jnp.ones((n, ctx_len), dtype=m_ref.dtype)
    o_ref[...] = jnp.concatenate([ones, m_ref[...]], axis=1)


def kernel(class_token_ids, class_attention_mask, embedding_table, context_vectors_text):
    n, tok_len = class_token_ids.shape
    ctx_len, d = context_vectors_text.shape
    seq = 1 + ctx_len + (tok_len - 1)
    rpw = n // NW

    ids_w = class_token_ids.T.reshape(tok_len, NW, rpw).transpose(1, 0, 2).reshape(-1)
    ctx8 = jnp.repeat(context_vectors_text, REP, axis=0)

    mesh = plsc.VectorSubcoreMesh(core_axis_name="c", subcore_axis_name="s")
    emb_t = pl.kernel(
        functools.partial(_emb_kernel_body, n, tok_len, ctx_len, d),
        out_type=jax.ShapeDtypeStruct((seq, n, d), jnp.float32),
        mesh=mesh,
        scratch_types=[
            pltpu.VMEM((tok_len * rpw,), jnp.int32),
            [pltpu.VMEM((CK, d), jnp.float32) for _ in range(NBUF)],
            pltpu.VMEM((REP * ctx_len, d), jnp.float32),
            [pltpu.SemaphoreType.DMA for _ in range(NBUF)],
            [pltpu.SemaphoreType.DMA for _ in range(NBUF)],
            pltpu.SemaphoreType.DMA,
        ],
        compiler_params=pltpu.CompilerParams(use_tc_tiling_on_sc=True),
    )(ids_w, embedding_table, ctx8)

    mask = pl.pallas_call(
        functools.partial(_mask_body, ctx_len=ctx_len),
        out_shape=jax.ShapeDtypeStruct((n, ctx_len + tok_len), class_attention_mask.dtype),
    )(class_attention_mask)

    return emb_t.transpose(1, 0, 2), mask


# speedup vs baseline: 1.0316x; 1.0078x over previous
import functools

import jax
import jax.numpy as jnp
from jax import lax
from jax.experimental import pallas as pl
from jax.experimental.pallas import tpu as pltpu
from jax.experimental.pallas import tpu_sc as plsc

NC = 2
NS = 16
NW = NC * NS
REP = 8
NBUF = 4
CK = 16


def _emb_kernel_body(n, tok_len, ctx_len, d,
                     idx_hbm, table_hbm, ctx8_hbm, out_hbm,
                     idx_v, gbufs, ctx8_v, gsems, wsems, csem):
    rpw = n // NW
    nck = rpw // CK
    nq = tok_len * nck
    ncw = ctx_len * (rpw // REP)
    nm = nq // NBUF
    cw_per = -(-ncw // nm)

    wid = lax.axis_index("s") * NC + lax.axis_index("c")
    c0 = wid * rpw

    pltpu.sync_copy(idx_hbm.at[pl.ds(wid * tok_len * rpw, tok_len * rpw)], idx_v)
    pltpu.sync_copy(ctx8_hbm, ctx8_v)

    def g_src(q):
        g = q // nck
        i = q % nck
        return table_hbm.at[idx_v.at[pl.ds(g * rpw + i * CK, CK)]]

    def w_dst(q):
        g = q // nck
        i = q % nck
        t = jnp.where(g == 0, 0, g + ctx_len)
        return out_hbm.at[t, pl.ds(c0 + i * CK, CK), :]

    def ctx_src(j):
        return ctx8_v.at[pl.ds((j // (rpw // REP)) * REP, REP)]

    def ctx_dst(j):
        k = j // (rpw // REP)
        i = j % (rpw // REP)
        return out_hbm.at[k + 1, pl.ds(c0 + i * REP, REP), :]

    for b in range(NBUF):
        pltpu.async_copy(g_src(b), gbufs[b], gsems[b])

    def outer(m, carry):
        q0 = m * NBUF
        for b in range(NBUF):
            q = q0 + b
            pltpu.make_async_copy(g_src(q), gbufs[b], gsems[b]).wait()
            pltpu.async_copy(gbufs[b], w_dst(q), wsems[b])
        for s in range(cw_per):
            j = m * cw_per + s

            @pl.when(j < ncw)
            def _():
                pltpu.async_copy(ctx_src(j), ctx_dst(j), csem)
        for b in range(NBUF):
            q = q0 + b
            qn = q + NBUF

            @pl.when(qn < nq)
            def _():
                pltpu.make_async_copy(gbufs[b], w_dst(q), wsems[b]).wait()
                pltpu.async_copy(g_src(qn), gbufs[b], gsems[b])
        return carry

    lax.fori_loop(0, nm, outer, 0)

    for b in range(NBUF):
        q = nq - NBUF + b
        pltpu.make_async_copy(gbufs[b], w_dst(q), wsems[b]).wait()

    def ctx_drain(j, carry):
        pltpu.make_async_copy(ctx_src(j), ctx_dst(j), csem).wait()
        return carry

    lax.fori_loop(0, ncw, ctx_drain, 0)


def _mask_body(m_ref, o_ref, *, ctx_len):
    n = m_ref.shape[0]
    ones = jnp.ones((n, ctx_len), dtype=m_ref.dtype)
    o_ref[...] = jnp.concatenate([ones, m_ref[...]], axis=1)


def kernel(class_token_ids, class_attention_mask, embedding_table, context_vectors_text):
    n, tok_len = class_token_ids.shape
    ctx_len, d = context_vectors_text.shape
    seq = 1 + ctx_len + (tok_len - 1)
    rpw = n // NW

    ids_w = class_token_ids.T.reshape(tok_len, NW, rpw).transpose(1, 0, 2).reshape(-1)
    ctx8 = jnp.repeat(context_vectors_text, REP, axis=0)

    mesh = plsc.VectorSubcoreMesh(core_axis_name="c", subcore_axis_name="s")
    emb_t = pl.kernel(
        functools.partial(_emb_kernel_body, n, tok_len, ctx_len, d),
        out_type=jax.ShapeDtypeStruct((seq, n, d), jnp.float32),
        mesh=mesh,
        scratch_types=[
            pltpu.VMEM((tok_len * rpw,), jnp.int32),
            [pltpu.VMEM((CK, d), jnp.float32) for _ in range(NBUF)],
            pltpu.VMEM((REP * ctx_len, d), jnp.float32),
            [pltpu.SemaphoreType.DMA for _ in range(NBUF)],
            [pltpu.SemaphoreType.DMA for _ in range(NBUF)],
            pltpu.SemaphoreType.DMA,
        ],
        compiler_params=pltpu.CompilerParams(use_tc_tiling_on_sc=True),
    )(ids_w, embedding_table, ctx8)

    mask = pl.pallas_call(
        functools.partial(_mask_body, ctx_len=ctx_len),
        out_shape=jax.ShapeDtypeStruct((n, ctx_len + tok_len), class_attention_mask.dtype),
    )(class_attention_mask)

    return emb_t.transpose(1, 0, 2), mask
